# Initial kernel scaffold; baseline (speedup 1.0000x reference)
import jax
import jax.numpy as jnp
from jax.experimental import pallas as pl


def kernel(x1, x2, ln1_s, ln1_b, wqk, wv, wo, rot, ln2_s, ln2_b, w1, b1, w2, b2):
    raise NotImplementedError("write your pallas kernel here")


# speedup vs baseline: 1.9532x; 1.9532x over previous
import math

import jax
import jax.numpy as jnp
from jax import lax
from jax.experimental import pallas as pl
from jax.experimental.pallas import tpu as pltpu
from jax.experimental.pallas import tpu_sc as plsc

_D = 768
_H = 12
_DH = 64
_F = 3072
_R = 2
_C = 64
_NB = 128
_S = 8192
_NT = _R * _H
_NTS = _NT * _S
_TS = 512
_NSUB = _TS // _C
_NCH = _S // _C

_f32 = jnp.float32


def _proj_body(x_ref, s_ref, b_ref, wqk_ref, wv_ref, qk_ref, v_ref):
    x = x_ref[...]
    m = jnp.mean(x, axis=-1, keepdims=True)
    var = jnp.mean((x - m) * (x - m), axis=-1, keepdims=True)
    h = (x - m) / jnp.sqrt(var + 1e-5) * s_ref[...][None, :] + b_ref[...][None, :]
    for hh in range(_H):
        qk_ref[hh] = h @ wqk_ref[:, hh * _DH:(hh + 1) * _DH]
        v_ref[hh] = h @ wv_ref[:, hh * _DH:(hh + 1) * _DH]


def _proj(x, s, b, wqk_l, wv_l):
    return pl.pallas_call(
        _proj_body,
        grid=(_S // _TS,),
        in_specs=[
            pl.BlockSpec((_TS, _D), lambda i: (i, 0)),
            pl.BlockSpec((_D,), lambda i: (0,)),
            pl.BlockSpec((_D,), lambda i: (0,)),
            pl.BlockSpec((_D, _D), lambda i: (0, 0)),
            pl.BlockSpec((_D, _D), lambda i: (0, 0)),
        ],
        out_specs=[
            pl.BlockSpec((_H, _TS, _DH), lambda i: (0, i, 0)),
            pl.BlockSpec((_H, _TS, _DH), lambda i: (0, i, 0)),
        ],
        out_shape=[jax.ShapeDtypeStruct((_H, _S, _DH), _f32)] * 2,
    )(x, s, b, wqk_l, wv_l)


def _bucket_body(qk_ref, rot_ref, out_ref):
    t = pl.program_id(0)
    qk = qk_ref[0]
    rr = rot_ref[0]
    proj = lax.dot_general(qk, rr, (((1,), (0,)), ((), ())))
    x = jnp.concatenate([proj, -proj], axis=-1)

    r1 = lax.broadcasted_iota(jnp.int32, (_NB, _NB), 0)
    c1 = lax.broadcasted_iota(jnp.int32, (_NB, _NB), 1)
    lincl = (r1 >= c1).astype(_f32)
    ustrict = (r1 < c1).astype(_f32)
    mx = jnp.max(x, axis=-1, keepdims=True)
    eq = (x >= mx).astype(_f32)
    nleft = lax.dot_general(eq, ustrict, (((1,), (0,)), ((), ())))
    oh = eq * (nleft == 0.0).astype(_f32)

    run = jnp.zeros((1, _NB), _f32)
    rows = []
    for c in range(_S // _NB):
        blk = oh[c * _NB:(c + 1) * _NB]
        incl = lax.dot_general(lincl, blk, (((1,), (0,)), ((), ())))
        rows.append(jnp.sum((incl + run) * blk, axis=-1) - 1.0)
        run = run + incl[-1:, :]
    starts = lax.dot_general(run, ustrict, (((1,), (0,)), ((), ())))
    for c in range(_S // _NB):
        blk = oh[c * _NB:(c + 1) * _NB]
        so = jnp.sum(starts * blk, axis=-1)
        out_ref[c] = (rows[c] + so).astype(jnp.int32) + t * _S


def _bucket(qkf, rot_l):
    return pl.pallas_call(
        _bucket_body,
        grid=(_NT,),
        in_specs=[
            pl.BlockSpec((1, _S, _DH), lambda t: (t % _H, 0, 0)),
            pl.BlockSpec((1, _DH, _NB // 2), lambda t: (t // _H, 0, 0)),
        ],
        out_specs=pl.BlockSpec((_S // _NB, _NB), lambda t: (t, 0)),
        out_shape=jax.ShapeDtypeStruct((_NT * _S // _NB, _NB), jnp.int32),
    )(qkf, rot_l)


def _sc_scatter_body(qk_hbm, v_hbm, gd_hbm, qktab, vtab, postab,
                     idxg, qkrows, vrows, posv, sem):
    wid = lax.axis_index("s") * 2 + lax.axis_index("c")
    for t in range(_NT):
        h = t % _H
        for j in range(2):
            cc = wid * 2 + j
            s0 = cc * 128
            g0 = t * _S + s0
            src0 = h * _S + s0
            pltpu.async_copy(gd_hbm.at[pl.ds(g0, 128)], idxg, sem).wait()
            pltpu.async_copy(qk_hbm.at[pl.ds(src0, 128)], qkrows, sem).wait()
            pltpu.async_copy(v_hbm.at[pl.ds(src0, 128)], vrows, sem).wait()
            for k in range(8):
                sl = pl.ds(k * 16, 16)
                posv[sl] = (lax.iota(jnp.int32, 16) + (s0 + k * 16)).astype(_f32)
            pltpu.async_copy(qkrows, qktab.at[idxg], sem).wait()
            pltpu.async_copy(vrows, vtab.at[idxg], sem).wait()
            pltpu.async_copy(posv, postab.at[idxg], sem).wait()


def _sc_scatter(qkflat, vflat, gd):
    return pl.kernel(
        _sc_scatter_body,
        out_type=(jax.ShapeDtypeStruct((_NTS, _DH), _f32),
                  jax.ShapeDtypeStruct((_NTS, _DH), _f32),
                  jax.ShapeDtypeStruct((_NTS,), _f32)),
        mesh=plsc.VectorSubcoreMesh(core_axis_name="c", subcore_axis_name="s"),
        scratch_types=[pltpu.VMEM((128,), jnp.int32),
                       pltpu.VMEM((128, _DH), _f32),
                       pltpu.VMEM((128, _DH), _f32),
                       pltpu.VMEM((128,), _f32),
                       pltpu.SemaphoreType.DMA],
        compiler_params=pltpu.CompilerParams(use_tc_tiling_on_sc=False),
    )(qkflat, vflat, gd)


def _attn_body(qc_ref, qp_ref, vc_ref, vp_ref, pc_ref, pp_ref, o_ref, l_ref):
    qc = qc_ref[0]
    kfull = jnp.concatenate([qp_ref[0], qc], axis=0)
    vfull = jnp.concatenate([vp_ref[0], vc_ref[0]], axis=0)
    pcur = pc_ref[0]
    pfull = jnp.concatenate([pp_ref[0], pcur], axis=0)
    r1 = lax.broadcasted_iota(jnp.int32, (2 * _C, 2 * _C), 0)
    c1 = lax.broadcasted_iota(jnp.int32, (2 * _C, 2 * _C), 1)
    eye = (r1 == c1).astype(_f32)
    scale = 1.0 / math.sqrt(_DH)
    outs = []
    lses = []
    for sub in range(_NSUB):
        q = qc[sub * _C:(sub + 1) * _C]
        kw = kfull[sub * _C: sub * _C + 2 * _C]
        vw = vfull[sub * _C: sub * _C + 2 * _C]
        kn = kw / (jnp.sqrt(jnp.sum(kw * kw, axis=-1, keepdims=True)) + 1e-6)
        pq = pcur[sub * _C:(sub + 1) * _C]
        pkc = pfull[sub * _C: sub * _C + 2 * _C]
        pk = lax.dot_general(pkc, eye, (((0,), (0,)), ((), ())))
        dots = lax.dot_general(q, kn, (((1,), (1,)), ((), ()))) * scale
        causal = pq >= pk
        selfm = pq == pk
        dots = jnp.where(causal, dots, -1e9)
        dots = jnp.where(selfm, dots - 1e5, dots)
        m = jnp.max(dots, axis=-1, keepdims=True)
        e = jnp.exp(dots - m)
        ssum = jnp.sum(e, axis=-1, keepdims=True)
        lses.append(m + jnp.log(ssum))
        outs.append(lax.dot_general(e / ssum, vw, (((1,), (0,)), ((), ()))))
    o_ref[0] = jnp.concatenate(outs, axis=0)
    l_ref[0] = jnp.concatenate(lses, axis=0)


def _attn(qktab, vtab, postab):
    prev_map = lambda t, i: (t, (i * _NSUB + _NCH - 1) % _NCH, 0)
    return pl.pallas_call(
        _attn_body,
        grid=(_NT, _S // _TS),
        in_specs=[
            pl.BlockSpec((1, _TS, _DH), lambda t, i: (t, i, 0)),
            pl.BlockSpec((1, _C, _DH), prev_map),
            pl.BlockSpec((1, _TS, _DH), lambda t, i: (t, i, 0)),
            pl.BlockSpec((1, _C, _DH), prev_map),
            pl.BlockSpec((1, _TS, 1), lambda t, i: (t, i, 0)),
            pl.BlockSpec((1, _C, 1), prev_map),
        ],
        out_specs=[
            pl.BlockSpec((1, _TS, _DH), lambda t, i: (t, i, 0)),
            pl.BlockSpec((1, _TS, 1), lambda t, i: (t, i, 0)),
        ],
        out_shape=[jax.ShapeDtypeStruct((_NT, _S, _DH), _f32),
                   jax.ShapeDtypeStruct((_NT, _S, 1), _f32)],
    )(qktab, qktab, vtab, vtab, postab, postab)


def _sc_gather_body(otab, ltab, gd_hbm, ou, lu, idxg, orows, lrow, sem):
    wid = lax.axis_index("s") * 2 + lax.axis_index("c")
    for t in range(_NT):
        for j in range(2):
            cc = wid * 2 + j
            g0 = t * _S + cc * 128
            pltpu.async_copy(gd_hbm.at[pl.ds(g0, 128)], idxg, sem).wait()
            pltpu.async_copy(otab.at[idxg], orows, sem).wait()
            pltpu.async_copy(ltab.at[idxg], lrow, sem).wait()
            pltpu.async_copy(orows, ou.at[pl.ds(g0, 128)], sem).wait()
            pltpu.async_copy(lrow, lu.at[pl.ds(g0, 128)], sem).wait()


def _sc_gather(oflat, lflat, gd):
    return pl.kernel(
        _sc_gather_body,
        out_type=(jax.ShapeDtypeStruct((_NTS, _DH), _f32),
                  jax.ShapeDtypeStruct((_NTS,), _f32)),
        mesh=plsc.VectorSubcoreMesh(core_axis_name="c", subcore_axis_name="s"),
        scratch_types=[pltpu.VMEM((128,), jnp.int32),
                       pltpu.VMEM((128, _DH), _f32),
                       pltpu.VMEM((128,), _f32),
                       pltpu.SemaphoreType.DMA],
        compiler_params=pltpu.CompilerParams(use_tc_tiling_on_sc=False),
    )(oflat, lflat, gd)


def _combine_body(o0_ref, o1_ref, l0_ref, l1_ref, x1_ref, wo_ref, y_ref):
    o0 = o0_ref[0]
    o1 = o1_ref[0]
    l0 = l0_ref[0]
    l1 = l1_ref[0]
    m = jnp.maximum(l0, l1)
    e0 = jnp.exp(l0 - m)
    e1 = jnp.exp(l1 - m)
    inv = 1.0 / (e0 + e1)
    acc = x1_ref[...]
    for hh in range(_H):
        ohh = (e0[hh] * inv[hh]) * o0[hh] + (e1[hh] * inv[hh]) * o1[hh]
        acc = acc + lax.dot_general(
            ohh, wo_ref[hh * _DH:(hh + 1) * _DH, :], (((1,), (0,)), ((), ())))
    y_ref[...] = acc


def _combine(ou4, lu4, x1c, wo_l):
    return pl.pallas_call(
        _combine_body,
        grid=(_S // _TS,),
        in_specs=[
            pl.BlockSpec((1, _H, _TS, _DH), lambda i: (0, 0, i, 0)),
            pl.BlockSpec((1, _H, _TS, _DH), lambda i: (1, 0, i, 0)),
            pl.BlockSpec((1, _H, _TS, 1), lambda i: (0, 0, i, 0)),
            pl.BlockSpec((1, _H, _TS, 1), lambda i: (1, 0, i, 0)),
            pl.BlockSpec((_TS, _D), lambda i: (i, 0)),
            pl.BlockSpec((_D, _D), lambda i: (0, 0)),
        ],
        out_specs=pl.BlockSpec((_TS, _D), lambda i: (i, 0)),
        out_shape=jax.ShapeDtypeStruct((_S, _D), _f32),
    )(ou4, ou4, lu4, lu4, x1c, wo_l)


def _ff_body(y1_ref, x2_ref, s_ref, b_ref, w1_ref, b1_ref, w2_ref, b2_ref,
             out_ref):
    x = y1_ref[...]
    m = jnp.mean(x, axis=-1, keepdims=True)
    var = jnp.mean((x - m) * (x - m), axis=-1, keepdims=True)
    h = (x - m) / jnp.sqrt(var + 1e-5) * s_ref[...][None, :] + b_ref[...][None, :]
    a = h @ w1_ref[...] + b1_ref[...][None, :]
    g = jax.nn.gelu(a)
    out_ref[...] = x2_ref[...] + g @ w2_ref[...] + b2_ref[...][None, :]


def _ff(y1, x2c, s, b, w1_l, b1_l, w2_l, b2_l):
    return pl.pallas_call(
        _ff_body,
        grid=(_S // _TS,),
        in_specs=[
            pl.BlockSpec((_TS, _D), lambda i: (i, 0)),
            pl.BlockSpec((_TS, _D), lambda i: (i, 0)),
            pl.BlockSpec((_D,), lambda i: (0,)),
            pl.BlockSpec((_D,), lambda i: (0,)),
            pl.BlockSpec((_D, _F), lambda i: (0, 0)),
            pl.BlockSpec((_F,), lambda i: (0,)),
            pl.BlockSpec((_F, _D), lambda i: (0, 0)),
            pl.BlockSpec((_D,), lambda i: (0,)),
        ],
        out_specs=pl.BlockSpec((_TS, _D), lambda i: (i, 0)),
        out_shape=jax.ShapeDtypeStruct((_S, _D), _f32),
    )(y1, x2c, s, b, w1_l, b1_l, w2_l, b2_l)


def kernel(x1, x2, ln1_s, ln1_b, wqk, wv, wo, rot, ln2_s, ln2_b, w1, b1, w2, b2):
    xa = x1[0]
    xb = x2[0]
    for l in range(2):
        qkf, vf = _proj(xb, ln1_s[l], ln1_b[l], wqk[l], wv[l])
        gd = _bucket(qkf, rot[l]).reshape(_NTS)
        qktab, vtab, postab = _sc_scatter(
            qkf.reshape(_H * _S, _DH), vf.reshape(_H * _S, _DH), gd)
        o, lse = _attn(qktab.reshape(_NT, _S, _DH),
                       vtab.reshape(_NT, _S, _DH),
                       postab.reshape(_NT, _S, 1))
        ou, lu = _sc_gather(o.reshape(_NTS, _DH), lse.reshape(_NTS), gd)
        ya = _combine(ou.reshape(_R, _H, _S, _DH),
                      lu.reshape(_R, _H, _S, 1), xa, wo[l])
        yb = _ff(ya, xb, ln2_s[l], ln2_b[l], w1[l], b1[l], w2[l], b2[l])
        xa, xb = ya, yb
    return xb[None]


# speedup vs baseline: 2.5557x; 1.3085x over previous
import math

import jax
import jax.numpy as jnp
from jax import lax
from jax.experimental import pallas as pl
from jax.experimental.pallas import tpu as pltpu
from jax.experimental.pallas import tpu_sc as plsc

_D = 768
_H = 12
_DH = 64
_F = 3072
_R = 2
_C = 64
_NB = 128
_S = 8192
_NT = _R * _H
_NTS = _NT * _S
_TS = 512
_NSUB = _TS // _C
_NCH = _S // _C
_NW = 32
_CPW = _NT * (_S // 128) // _NW

_f32 = jnp.float32


def _proj_body(x_ref, s_ref, b_ref, wqk_ref, wv_ref, qv_ref):
    x = x_ref[...]
    m = jnp.mean(x, axis=-1, keepdims=True)
    var = jnp.mean((x - m) * (x - m), axis=-1, keepdims=True)
    h = (x - m) / jnp.sqrt(var + 1e-5) * s_ref[...][None, :] + b_ref[...][None, :]
    for hh in range(_H):
        sl = slice(hh * _DH, (hh + 1) * _DH)
        qv_ref[hh] = jnp.concatenate([h @ wqk_ref[:, sl], h @ wv_ref[:, sl]],
                                     axis=-1)


def _proj(x, s, b, wqk_l, wv_l):
    return pl.pallas_call(
        _proj_body,
        grid=(_S // _TS,),
        in_specs=[
            pl.BlockSpec((_TS, _D), lambda i: (i, 0)),
            pl.BlockSpec((_D,), lambda i: (0,)),
            pl.BlockSpec((_D,), lambda i: (0,)),
            pl.BlockSpec((_D, _D), lambda i: (0, 0)),
            pl.BlockSpec((_D, _D), lambda i: (0, 0)),
        ],
        out_specs=pl.BlockSpec((_H, _TS, 2 * _DH), lambda i: (0, i, 0)),
        out_shape=jax.ShapeDtypeStruct((_H, _S, 2 * _DH), _f32),
    )(x, s, b, wqk_l, wv_l)


def _bucket_body(qv_ref, rot_ref, out_ref):
    t = pl.program_id(0)
    qk = qv_ref[0][:, :_DH]
    rr = rot_ref[0]
    proj = lax.dot_general(qk, rr, (((1,), (0,)), ((), ())))
    x = jnp.concatenate([proj, -proj], axis=-1)

    r1 = lax.broadcasted_iota(jnp.int32, (_NB, _NB), 0)
    c1 = lax.broadcasted_iota(jnp.int32, (_NB, _NB), 1)
    lincl = (r1 >= c1).astype(_f32)
    ustrict = (r1 < c1).astype(_f32)
    mx = jnp.max(x, axis=-1, keepdims=True)
    eq = (x >= mx).astype(_f32)
    nleft = lax.dot_general(eq, ustrict, (((1,), (0,)), ((), ())))
    oh = eq * (nleft == 0.0).astype(_f32)

    nch = _S // _NB
    totals = jnp.sum(oh.reshape(nch, _NB, _NB), axis=1)
    r2 = lax.broadcasted_iota(jnp.int32, (nch, nch), 0)
    c2 = lax.broadcasted_iota(jnp.int32, (nch, nch), 1)
    lstrict = (r2 > c2).astype(_f32)
    chunk_excl = lax.dot_general(lstrict, totals, (((1,), (0,)), ((), ())))
    gcounts = chunk_excl[-1:] + totals[-1:]
    starts = lax.dot_general(gcounts, ustrict, (((1,), (0,)), ((), ())))
    base = starts + chunk_excl
    for c in range(nch):
        blk = oh[c * _NB:(c + 1) * _NB]
        incl = lax.dot_general(lincl, blk, (((1,), (0,)), ((), ())))
        dest_c = jnp.sum((incl + base[c:c + 1]) * blk, axis=-1) - 1.0
        out_ref[c] = dest_c.astype(jnp.int32) + t * _S


def _bucket(qv, rot_l):
    return pl.pallas_call(
        _bucket_body,
        grid=(_NT,),
        in_specs=[
            pl.BlockSpec((1, _S, 2 * _DH), lambda t: (t % _H, 0, 0)),
            pl.BlockSpec((1, _DH, _NB // 2), lambda t: (t // _H, 0, 0)),
        ],
        out_specs=pl.BlockSpec((_S // _NB, _NB), lambda t: (t, 0)),
        out_shape=jax.ShapeDtypeStruct((_NT * _S // _NB, _NB), jnp.int32),
    )(qv, rot_l)


def _sc_scatter_body(qv_hbm, gd_hbm, tab, postab,
                     idx0, idx1, idx2, idx3,
                     row0, row1, row2, row3,
                     pos0, pos1, pos2, pos3,
                     ss0, ss1, ss2, ss3, sc0, sc1, sc2, sc3):
    idxb = [idx0, idx1, idx2, idx3]
    rowb = [row0, row1, row2, row3]
    posb = [pos0, pos1, pos2, pos3]
    sstg = [ss0, ss1, ss2, ss3]
    ssct = [sc0, sc1, sc2, sc3]
    wid = lax.axis_index("s") * 2 + lax.axis_index("c")

    def chunk_addr(i):
        t = i // 2
        cc = wid * 2 + (i % 2)
        s0 = cc * 128
        return t * _S + s0, (t % _H) * _S + s0, s0

    stage_d = {}
    scat_d = {}

    def fire_stage(i):
        slot = i % 4
        g0, src0, s0 = chunk_addr(i)
        d1 = pltpu.async_copy(gd_hbm.at[pl.ds(g0, 128)], idxb[slot], sstg[slot])
        d2 = pltpu.async_copy(qv_hbm.at[pl.ds(src0, 128)], rowb[slot], sstg[slot])
        for k in range(8):
            sl = pl.ds(k * 16, 16)
            posb[slot][sl] = (lax.iota(jnp.int32, 16) + (s0 + k * 16)).astype(_f32)
        stage_d[i] = (d1, d2)

    fire_stage(0)
    fire_stage(1)
    for i in range(_CPW):
        slot = i % 4
        if i + 2 < _CPW:
            if i - 2 >= 0:
                for d in scat_d.pop(i - 2):
                    d.wait()
            fire_stage(i + 2)
        for d in stage_d.pop(i):
            d.wait()
        d3 = pltpu.async_copy(rowb[slot], tab.at[idxb[slot]], ssct[slot])
        d4 = pltpu.async_copy(posb[slot], postab.at[idxb[slot]], ssct[slot])
        scat_d[i] = (d3, d4)
    for i in sorted(scat_d):
        for d in scat_d[i]:
            d.wait()


def _sc_scatter(qvflat, gd):
    sem = pltpu.SemaphoreType.DMA
    return pl.kernel(
        _sc_scatter_body,
        out_type=(jax.ShapeDtypeStruct((_NTS, 2 * _DH), _f32),
                  jax.ShapeDtypeStruct((_NTS,), _f32)),
        mesh=plsc.VectorSubcoreMesh(core_axis_name="c", subcore_axis_name="s"),
        scratch_types=[pltpu.VMEM((128,), jnp.int32)] * 4
                      + [pltpu.VMEM((128, 2 * _DH), _f32)] * 4
                      + [pltpu.VMEM((128,), _f32)] * 4
                      + [sem] * 8,
        compiler_params=pltpu.CompilerParams(use_tc_tiling_on_sc=False),
    )(qvflat, gd)


def _attn_body(qc_ref, qp_ref, pc_ref, pp_ref, o_ref):
    cur = qc_ref[0]
    full = jnp.concatenate([qp_ref[0], cur], axis=0)
    pcur = pc_ref[0]
    pfull = jnp.concatenate([pp_ref[0], pcur], axis=0)
    r1 = lax.broadcasted_iota(jnp.int32, (2 * _C, 2 * _C), 0)
    c1 = lax.broadcasted_iota(jnp.int32, (2 * _C, 2 * _C), 1)
    eye = (r1 == c1).astype(_f32)
    scale = 1.0 / math.sqrt(_DH)
    outs = []
    lses = []
    for sub in range(_NSUB):
        q = cur[sub * _C:(sub + 1) * _C, :_DH]
        win = full[sub * _C: sub * _C + 2 * _C]
        kw = win[:, :_DH]
        vw = win[:, _DH:]
        kn = kw / (jnp.sqrt(jnp.sum(kw * kw, axis=-1, keepdims=True)) + 1e-6)
        pq = pcur[sub * _C:(sub + 1) * _C]
        pkc = pfull[sub * _C: sub * _C + 2 * _C]
        pk = lax.dot_general(pkc, eye, (((0,), (0,)), ((), ())))
        dots = lax.dot_general(q, kn, (((1,), (1,)), ((), ()))) * scale
        causal = pq >= pk
        selfm = pq == pk
        dots = jnp.where(causal, dots, -1e9)
        dots = jnp.where(selfm, dots - 1e5, dots)
        m = jnp.max(dots, axis=-1, keepdims=True)
        e = jnp.exp(dots - m)
        ssum = jnp.sum(e, axis=-1, keepdims=True)
        lses.append(m + jnp.log(ssum))
        outs.append(lax.dot_general(e / ssum, vw, (((1,), (0,)), ((), ()))))
    pad = jnp.zeros((_TS, 2 * _DH - _DH - 1), _f32)
    o_ref[0] = jnp.concatenate(
        [jnp.concatenate(outs, axis=0), jnp.concatenate(lses, axis=0), pad],
        axis=-1)


def _attn(qvtab, postab):
    prev_map = lambda t, i: (t, (i * _NSUB + _NCH - 1) % _NCH, 0)
    return pl.pallas_call(
        _attn_body,
        grid=(_NT, _S // _TS),
        in_specs=[
            pl.BlockSpec((1, _TS, 2 * _DH), lambda t, i: (t, i, 0)),
            pl.BlockSpec((1, _C, 2 * _DH), prev_map),
            pl.BlockSpec((1, _TS, 1), lambda t, i: (t, i, 0)),
            pl.BlockSpec((1, _C, 1), prev_map),
        ],
        out_specs=pl.BlockSpec((1, _TS, 2 * _DH), lambda t, i: (t, i, 0)),
        out_shape=jax.ShapeDtypeStruct((_NT, _S, 2 * _DH), _f32),
    )(qvtab, qvtab, postab, postab)


def _sc_gather_body(otab, gd_hbm, ou,
                    idx0, idx1, idx2, idx3,
                    row0, row1, row2, row3,
                    si0, si1, si2, si3, sg0, sg1, sg2, sg3,
                    sw0, sw1, sw2, sw3):
    idxb = [idx0, idx1, idx2, idx3]
    rowb = [row0, row1, row2, row3]
    sidx = [si0, si1, si2, si3]
    sgat = [sg0, sg1, sg2, sg3]
    swrt = [sw0, sw1, sw2, sw3]
    wid = lax.axis_index("s") * 2 + lax.axis_index("c")

    def g_of(i):
        return (i // 2) * _S + (wid * 2 + (i % 2)) * 128

    idx_d = {}
    gat_d = {}
    wrt_d = {}

    def fire_idx(i):
        slot = i % 4
        idx_d[i] = pltpu.async_copy(gd_hbm.at[pl.ds(g_of(i), 128)],
                                    idxb[slot], sidx[slot])

    fire_idx(0)
    fire_idx(1)
    for i in range(_CPW):
        slot = i % 4
        if i + 2 < _CPW:
            if i - 2 >= 0:
                wrt_d.pop(i - 2).wait()
            fire_idx(i + 2)
        idx_d.pop(i).wait()
        gat_d[i] = pltpu.async_copy(otab.at[idxb[slot]], rowb[slot], sgat[slot])
        if i - 1 >= 0:
            gat_d.pop(i - 1).wait()
            ps = (i - 1) % 4
            wrt_d[i - 1] = pltpu.async_copy(rowb[ps], ou.at[pl.ds(g_of(i - 1), 128)],
                                            swrt[ps])
    gat_d.pop(_CPW - 1).wait()
    ls = (_CPW - 1) % 4
    wrt_d[_CPW - 1] = pltpu.async_copy(rowb[ls], ou.at[pl.ds(g_of(_CPW - 1), 128)],
                                       swrt[ls])
    for i in sorted(wrt_d):
        wrt_d[i].wait()


def _sc_gather(oflat, gd):
    sem = pltpu.SemaphoreType.DMA
    return pl.kernel(
        _sc_gather_body,
        out_type=jax.ShapeDtypeStruct((_NTS, 2 * _DH), _f32),
        mesh=plsc.VectorSubcoreMesh(core_axis_name="c", subcore_axis_name="s"),
        scratch_types=[pltpu.VMEM((128,), jnp.int32)] * 4
                      + [pltpu.VMEM((128, 2 * _DH), _f32)] * 4
                      + [sem] * 12,
        compiler_params=pltpu.CompilerParams(use_tc_tiling_on_sc=False),
    )(oflat, gd)


def _combine_body(o0_ref, o1_ref, x1_ref, wo_ref, y_ref):
    a0 = o0_ref[0]
    a1 = o1_ref[0]
    l0 = a0[..., _DH:_DH + 1]
    l1 = a1[..., _DH:_DH + 1]
    m = jnp.maximum(l0, l1)
    e0 = jnp.exp(l0 - m)
    e1 = jnp.exp(l1 - m)
    inv = 1.0 / (e0 + e1)
    acc = x1_ref[...]
    for hh in range(_H):
        ohh = ((e0[hh] * inv[hh]) * a0[hh, :, :_DH]
               + (e1[hh] * inv[hh]) * a1[hh, :, :_DH])
        acc = acc + lax.dot_general(
            ohh, wo_ref[hh * _DH:(hh + 1) * _DH, :], (((1,), (0,)), ((), ())))
    y_ref[...] = acc


def _combine(ou4, x1c, wo_l):
    return pl.pallas_call(
        _combine_body,
        grid=(_S // _TS,),
        in_specs=[
            pl.BlockSpec((1, _H, _TS, 2 * _DH), lambda i: (0, 0, i, 0)),
            pl.BlockSpec((1, _H, _TS, 2 * _DH), lambda i: (1, 0, i, 0)),
            pl.BlockSpec((_TS, _D), lambda i: (i, 0)),
            pl.BlockSpec((_D, _D), lambda i: (0, 0)),
        ],
        out_specs=pl.BlockSpec((_TS, _D), lambda i: (i, 0)),
        out_shape=jax.ShapeDtypeStruct((_S, _D), _f32),
    )(ou4, ou4, x1c, wo_l)


def _ff_body(y1_ref, x2_ref, s_ref, b_ref, w1_ref, b1_ref, w2_ref, b2_ref,
             out_ref):
    x = y1_ref[...]
    m = jnp.mean(x, axis=-1, keepdims=True)
    var = jnp.mean((x - m) * (x - m), axis=-1, keepdims=True)
    h = (x - m) / jnp.sqrt(var + 1e-5) * s_ref[...][None, :] + b_ref[...][None, :]
    a = h @ w1_ref[...] + b1_ref[...][None, :]
    g = jax.nn.gelu(a)
    out_ref[...] = x2_ref[...] + g @ w2_ref[...] + b2_ref[...][None, :]


def _ff(y1, x2c, s, b, w1_l, b1_l, w2_l, b2_l):
    return pl.pallas_call(
        _ff_body,
        grid=(_S // _TS,),
        in_specs=[
            pl.BlockSpec((_TS, _D), lambda i: (i, 0)),
            pl.BlockSpec((_TS, _D), lambda i: (i, 0)),
            pl.BlockSpec((_D,), lambda i: (0,)),
            pl.BlockSpec((_D,), lambda i: (0,)),
            pl.BlockSpec((_D, _F), lambda i: (0, 0)),
            pl.BlockSpec((_F,), lambda i: (0,)),
            pl.BlockSpec((_F, _D), lambda i: (0, 0)),
            pl.BlockSpec((_D,), lambda i: (0,)),
        ],
        out_specs=pl.BlockSpec((_TS, _D), lambda i: (i, 0)),
        out_shape=jax.ShapeDtypeStruct((_S, _D), _f32),
    )(y1, x2c, s, b, w1_l, b1_l, w2_l, b2_l)


def kernel(x1, x2, ln1_s, ln1_b, wqk, wv, wo, rot, ln2_s, ln2_b, w1, b1, w2, b2):
    xa = x1[0]
    xb = x2[0]
    for l in range(2):
        qv = _proj(xb, ln1_s[l], ln1_b[l], wqk[l], wv[l])
        gd = _bucket(qv, rot[l]).reshape(_NTS)
        qvtab, postab = _sc_scatter(qv.reshape(_H * _S, 2 * _DH), gd)
        olu = _attn(qvtab.reshape(_NT, _S, 2 * _DH), postab.reshape(_NT, _S, 1))
        ou = _sc_gather(olu.reshape(_NTS, 2 * _DH), gd)
        ya = _combine(ou.reshape(_R, _H, _S, 2 * _DH), xa, wo[l])
        yb = _ff(ya, xb, ln2_s[l], ln2_b[l], w1[l], b1[l], w2[l], b2[l])
        xa, xb = ya, yb
    return xb[None]


# speedup vs baseline: 2.9707x; 1.1624x over previous
import math

import jax
import jax.numpy as jnp
from jax import lax
from jax.experimental import pallas as pl
from jax.experimental.pallas import tpu as pltpu
from jax.experimental.pallas import tpu_sc as plsc

_D = 768
_H = 12
_DH = 64
_F = 3072
_R = 2
_C = 64
_NB = 128
_S = 8192
_NT = _R * _H
_NTS = _NT * _S
_TS = 512
_NSUB = _TS // _C
_NCH = _S // _C
_NW = 32
_CPW = _NT * (_S // 128) // _NW

_f32 = jnp.float32


def _proj_body(x_ref, s_ref, b_ref, wqk_ref, wv_ref, qv_ref):
    x = x_ref[...]
    m = jnp.mean(x, axis=-1, keepdims=True)
    var = jnp.mean((x - m) * (x - m), axis=-1, keepdims=True)
    h = (x - m) / jnp.sqrt(var + 1e-5) * s_ref[...][None, :] + b_ref[...][None, :]
    for hh in range(_H):
        sl = slice(hh * _DH, (hh + 1) * _DH)
        qv_ref[hh] = jnp.concatenate([h @ wqk_ref[:, sl], h @ wv_ref[:, sl]],
                                     axis=-1)


def _proj(x, s, b, wqk_l, wv_l):
    return pl.pallas_call(
        _proj_body,
        grid=(_S // _TS,),
        in_specs=[
            pl.BlockSpec((_TS, _D), lambda i: (i, 0)),
            pl.BlockSpec((_D,), lambda i: (0,)),
            pl.BlockSpec((_D,), lambda i: (0,)),
            pl.BlockSpec((_D, _D), lambda i: (0, 0)),
            pl.BlockSpec((_D, _D), lambda i: (0, 0)),
        ],
        out_specs=pl.BlockSpec((_H, _TS, 2 * _DH), lambda i: (0, i, 0)),
        out_shape=jax.ShapeDtypeStruct((_H, _S, 2 * _DH), _f32),
    )(x, s, b, wqk_l, wv_l)


def _bucket_body(qv_ref, rot_ref, out_ref):
    t = pl.program_id(0)
    qk = qv_ref[0][:, :_DH]
    rr = rot_ref[0]
    proj = lax.dot_general(qk, rr, (((1,), (0,)), ((), ())))
    x = jnp.concatenate([proj, -proj], axis=-1)

    r1 = lax.broadcasted_iota(jnp.int32, (_NB, _NB), 0)
    c1 = lax.broadcasted_iota(jnp.int32, (_NB, _NB), 1)
    lincl = (r1 >= c1).astype(_f32)
    ustrict = (r1 < c1).astype(_f32)
    mx = jnp.max(x, axis=-1, keepdims=True)
    eq = (x >= mx).astype(_f32)
    nleft = lax.dot_general(eq, ustrict, (((1,), (0,)), ((), ())))
    oh = eq * (nleft == 0.0).astype(_f32)

    nch = _S // _NB
    totals = jnp.sum(oh.reshape(nch, _NB, _NB), axis=1)
    r2 = lax.broadcasted_iota(jnp.int32, (nch, nch), 0)
    c2 = lax.broadcasted_iota(jnp.int32, (nch, nch), 1)
    lstrict = (r2 > c2).astype(_f32)
    chunk_excl = lax.dot_general(lstrict, totals, (((1,), (0,)), ((), ())))
    gcounts = chunk_excl[-1:] + totals[-1:]
    starts = lax.dot_general(gcounts, ustrict, (((1,), (0,)), ((), ())))
    base = starts + chunk_excl
    for c in range(nch):
        blk = oh[c * _NB:(c + 1) * _NB]
        incl = lax.dot_general(lincl, blk, (((1,), (0,)), ((), ())))
        dest_c = jnp.sum((incl + base[c:c + 1]) * blk, axis=-1) - 1.0
        out_ref[c] = dest_c.astype(jnp.int32) + t * _S


def _bucket(qv, rot_l):
    return pl.pallas_call(
        _bucket_body,
        grid=(_NT,),
        in_specs=[
            pl.BlockSpec((1, _S, 2 * _DH), lambda t: (t % _H, 0, 0)),
            pl.BlockSpec((1, _DH, _NB // 2), lambda t: (t // _H, 0, 0)),
        ],
        out_specs=pl.BlockSpec((_S // _NB, _NB), lambda t: (t, 0)),
        out_shape=jax.ShapeDtypeStruct((_NT * _S // _NB, _NB), jnp.int32),
    )(qv, rot_l)


def _sc_scatter_body(qv_hbm, gd_hbm, tab, postab,
                     idx0, idx1, idx2, idx3,
                     row0, row1, row2, row3,
                     pos0, pos1, pos2, pos3,
                     ss0, ss1, ss2, ss3, sc0, sc1, sc2, sc3):
    idxb = [idx0, idx1, idx2, idx3]
    rowb = [row0, row1, row2, row3]
    posb = [pos0, pos1, pos2, pos3]
    sstg = [ss0, ss1, ss2, ss3]
    ssct = [sc0, sc1, sc2, sc3]
    wid = lax.axis_index("s") * 2 + lax.axis_index("c")

    def chunk_addr(i):
        t = i // 2
        cc = wid * 2 + (i % 2)
        s0 = cc * 128
        return t * _S + s0, (t % _H) * _S + s0, s0

    stage_d = {}
    scat_d = {}

    def fire_stage(i):
        slot = i % 4
        g0, src0, s0 = chunk_addr(i)
        d1 = pltpu.async_copy(gd_hbm.at[pl.ds(g0, 128)], idxb[slot], sstg[slot])
        d2 = pltpu.async_copy(qv_hbm.at[pl.ds(src0, 128)], rowb[slot], sstg[slot])
        for k in range(8):
            sl = pl.ds(k * 16, 16)
            posb[slot][sl] = (lax.iota(jnp.int32, 16) + (s0 + k * 16)).astype(_f32)
        stage_d[i] = (d1, d2)

    fire_stage(0)
    fire_stage(1)
    for i in range(_CPW):
        slot = i % 4
        if i + 2 < _CPW:
            if i - 2 >= 0:
                for d in scat_d.pop(i - 2):
                    d.wait()
            fire_stage(i + 2)
        for d in stage_d.pop(i):
            d.wait()
        d3 = pltpu.async_copy(rowb[slot], tab.at[idxb[slot]], ssct[slot])
        d4 = pltpu.async_copy(posb[slot], postab.at[idxb[slot]], ssct[slot])
        scat_d[i] = (d3, d4)
    for i in sorted(scat_d):
        for d in scat_d[i]:
            d.wait()


def _sc_scatter(qvflat, gd):
    sem = pltpu.SemaphoreType.DMA
    return pl.kernel(
        _sc_scatter_body,
        out_type=(jax.ShapeDtypeStruct((_NTS, 2 * _DH), _f32),
                  jax.ShapeDtypeStruct((_NTS,), _f32)),
        mesh=plsc.VectorSubcoreMesh(core_axis_name="c", subcore_axis_name="s"),
        scratch_types=[pltpu.VMEM((128,), jnp.int32)] * 4
                      + [pltpu.VMEM((128, 2 * _DH), _f32)] * 4
                      + [pltpu.VMEM((128,), _f32)] * 4
                      + [sem] * 8,
        compiler_params=pltpu.CompilerParams(use_tc_tiling_on_sc=False),
    )(qvflat, gd)


def _attn_body(qc_ref, qp_ref, pc_ref, pp_ref, o_ref):
    cur = qc_ref[0]
    full = jnp.concatenate([qp_ref[0], cur], axis=0)
    kw = full[:, :_DH]
    vw = full[:, _DH:]
    kn = kw / (jnp.sqrt(jnp.sum(kw * kw, axis=-1, keepdims=True)) + 1e-6)
    q = cur[:, :_DH] * (1.0 / math.sqrt(_DH))
    pq = pc_ref[0]
    w = _TS + _C
    pfull = jnp.concatenate([pp_ref[0], pq], axis=0)
    r0 = lax.broadcasted_iota(jnp.int32, (w, w), 0)
    c0 = lax.broadcasted_iota(jnp.int32, (w, w), 1)
    eye = (r0 == c0).astype(_f32)
    pk = lax.dot_general(pfull, eye, (((0,), (0,)), ((), ())))
    sdiv = lax.broadcasted_iota(jnp.int32, (_TS, w), 0) // _C
    jdiv = lax.broadcasted_iota(jnp.int32, (_TS, w), 1) // _C
    win = (jdiv == sdiv) | (jdiv == sdiv + 1)
    dots = lax.dot_general(q, kn, (((1,), (1,)), ((), ())))
    dots = jnp.where(win & (pq >= pk), dots, -1e9)
    dots = jnp.where(win & (pq == pk), dots - 1e5, dots)
    m = jnp.max(dots, axis=-1, keepdims=True)
    e = jnp.exp(dots - m)
    ssum = jnp.sum(e, axis=-1, keepdims=True)
    lse = m + jnp.log(ssum)
    o = lax.dot_general(e / ssum, vw, (((1,), (0,)), ((), ())))
    pad = jnp.zeros((_TS, 2 * _DH - _DH - 1), _f32)
    o_ref[0] = jnp.concatenate([o, lse, pad], axis=-1)


def _attn(qvtab, postab):
    prev_map = lambda t, i: (t, (i * _NSUB + _NCH - 1) % _NCH, 0)
    return pl.pallas_call(
        _attn_body,
        grid=(_NT, _S // _TS),
        in_specs=[
            pl.BlockSpec((1, _TS, 2 * _DH), lambda t, i: (t, i, 0)),
            pl.BlockSpec((1, _C, 2 * _DH), prev_map),
            pl.BlockSpec((1, _TS, 1), lambda t, i: (t, i, 0)),
            pl.BlockSpec((1, _C, 1), prev_map),
        ],
        out_specs=pl.BlockSpec((1, _TS, 2 * _DH), lambda t, i: (t, i, 0)),
        out_shape=jax.ShapeDtypeStruct((_NT, _S, 2 * _DH), _f32),
    )(qvtab, qvtab, postab, postab)


def _sc_gather_body(otab, gd_hbm, ou,
                    idx0, idx1, idx2, idx3,
                    row0, row1, row2, row3,
                    si0, si1, si2, si3, sg0, sg1, sg2, sg3,
                    sw0, sw1, sw2, sw3):
    idxb = [idx0, idx1, idx2, idx3]
    rowb = [row0, row1, row2, row3]
    sidx = [si0, si1, si2, si3]
    sgat = [sg0, sg1, sg2, sg3]
    swrt = [sw0, sw1, sw2, sw3]
    wid = lax.axis_index("s") * 2 + lax.axis_index("c")

    def g_of(i):
        return (i // 2) * _S + (wid * 2 + (i % 2)) * 128

    idx_d = {}
    gat_d = {}
    wrt_d = {}

    def fire_idx(i):
        slot = i % 4
        idx_d[i] = pltpu.async_copy(gd_hbm.at[pl.ds(g_of(i), 128)],
                                    idxb[slot], sidx[slot])

    fire_idx(0)
    fire_idx(1)
    for i in range(_CPW):
        slot = i % 4
        if i + 2 < _CPW:
            if i - 2 >= 0:
                wrt_d.pop(i - 2).wait()
            fire_idx(i + 2)
        idx_d.pop(i).wait()
        gat_d[i] = pltpu.async_copy(otab.at[idxb[slot]], rowb[slot], sgat[slot])
        if i - 1 >= 0:
            gat_d.pop(i - 1).wait()
            ps = (i - 1) % 4
            wrt_d[i - 1] = pltpu.async_copy(rowb[ps], ou.at[pl.ds(g_of(i - 1), 128)],
                                            swrt[ps])
    gat_d.pop(_CPW - 1).wait()
    ls = (_CPW - 1) % 4
    wrt_d[_CPW - 1] = pltpu.async_copy(rowb[ls], ou.at[pl.ds(g_of(_CPW - 1), 128)],
                                       swrt[ls])
    for i in sorted(wrt_d):
        wrt_d[i].wait()


def _sc_gather(oflat, gd):
    sem = pltpu.SemaphoreType.DMA
    return pl.kernel(
        _sc_gather_body,
        out_type=jax.ShapeDtypeStruct((_NTS, 2 * _DH), _f32),
        mesh=plsc.VectorSubcoreMesh(core_axis_name="c", subcore_axis_name="s"),
        scratch_types=[pltpu.VMEM((128,), jnp.int32)] * 4
                      + [pltpu.VMEM((128, 2 * _DH), _f32)] * 4
                      + [sem] * 12,
        compiler_params=pltpu.CompilerParams(use_tc_tiling_on_sc=False),
    )(oflat, gd)


def _combine_body(o0_ref, o1_ref, x1_ref, wo_ref, y_ref):
    a0 = o0_ref[0]
    a1 = o1_ref[0]
    l0 = a0[..., _DH:_DH + 1]
    l1 = a1[..., _DH:_DH + 1]
    m = jnp.maximum(l0, l1)
    e0 = jnp.exp(l0 - m)
    e1 = jnp.exp(l1 - m)
    inv = 1.0 / (e0 + e1)
    acc = x1_ref[...]
    for hh in range(_H):
        ohh = ((e0[hh] * inv[hh]) * a0[hh, :, :_DH]
               + (e1[hh] * inv[hh]) * a1[hh, :, :_DH])
        acc = acc + lax.dot_general(
            ohh, wo_ref[hh * _DH:(hh + 1) * _DH, :], (((1,), (0,)), ((), ())))
    y_ref[...] = acc


def _combine(ou4, x1c, wo_l):
    return pl.pallas_call(
        _combine_body,
        grid=(_S // _TS,),
        in_specs=[
            pl.BlockSpec((1, _H, _TS, 2 * _DH), lambda i: (0, 0, i, 0)),
            pl.BlockSpec((1, _H, _TS, 2 * _DH), lambda i: (1, 0, i, 0)),
            pl.BlockSpec((_TS, _D), lambda i: (i, 0)),
            pl.BlockSpec((_D, _D), lambda i: (0, 0)),
        ],
        out_specs=pl.BlockSpec((_TS, _D), lambda i: (i, 0)),
        out_shape=jax.ShapeDtypeStruct((_S, _D), _f32),
    )(ou4, ou4, x1c, wo_l)


def _ff_body(y1_ref, x2_ref, s_ref, b_ref, w1_ref, b1_ref, w2_ref, b2_ref,
             out_ref):
    x = y1_ref[...]
    m = jnp.mean(x, axis=-1, keepdims=True)
    var = jnp.mean((x - m) * (x - m), axis=-1, keepdims=True)
    h = (x - m) / jnp.sqrt(var + 1e-5) * s_ref[...][None, :] + b_ref[...][None, :]
    a = h @ w1_ref[...] + b1_ref[...][None, :]
    g = jax.nn.gelu(a)
    out_ref[...] = x2_ref[...] + g @ w2_ref[...] + b2_ref[...][None, :]


def _ff(y1, x2c, s, b, w1_l, b1_l, w2_l, b2_l):
    return pl.pallas_call(
        _ff_body,
        grid=(_S // _TS,),
        in_specs=[
            pl.BlockSpec((_TS, _D), lambda i: (i, 0)),
            pl.BlockSpec((_TS, _D), lambda i: (i, 0)),
            pl.BlockSpec((_D,), lambda i: (0,)),
            pl.BlockSpec((_D,), lambda i: (0,)),
            pl.BlockSpec((_D, _F), lambda i: (0, 0)),
            pl.BlockSpec((_F,), lambda i: (0,)),
            pl.BlockSpec((_F, _D), lambda i: (0, 0)),
            pl.BlockSpec((_D,), lambda i: (0,)),
        ],
        out_specs=pl.BlockSpec((_TS, _D), lambda i: (i, 0)),
        out_shape=jax.ShapeDtypeStruct((_S, _D), _f32),
    )(y1, x2c, s, b, w1_l, b1_l, w2_l, b2_l)


def kernel(x1, x2, ln1_s, ln1_b, wqk, wv, wo, rot, ln2_s, ln2_b, w1, b1, w2, b2):
    xa = x1[0]
    xb = x2[0]
    for l in range(2):
        qv = _proj(xb, ln1_s[l], ln1_b[l], wqk[l], wv[l])
        gd = _bucket(qv, rot[l]).reshape(_NTS)
        qvtab, postab = _sc_scatter(qv.reshape(_H * _S, 2 * _DH), gd)
        olu = _attn(qvtab.reshape(_NT, _S, 2 * _DH), postab.reshape(_NT, _S, 1))
        ou = _sc_gather(olu.reshape(_NTS, 2 * _DH), gd)
        ya = _combine(ou.reshape(_R, _H, _S, 2 * _DH), xa, wo[l])
        yb = _ff(ya, xb, ln2_s[l], ln2_b[l], w1[l], b1[l], w2[l], b2[l])
        xa, xb = ya, yb
    return xb[None]


# speedup vs baseline: 2.9752x; 1.0015x over previous
import math

import jax
import jax.numpy as jnp
from jax import lax
from jax.experimental import pallas as pl
from jax.experimental.pallas import tpu as pltpu
from jax.experimental.pallas import tpu_sc as plsc

_D = 768
_H = 12
_DH = 64
_F = 3072
_R = 2
_C = 64
_NB = 128
_S = 8192
_NT = _R * _H
_NTS = _NT * _S
_TS = 512
_NSUB = _TS // _C
_NCH = _S // _C
_NW = 32
_CPW = _NT * (_S // 128) // _NW

_f32 = jnp.float32


def _proj_body(x_ref, s_ref, b_ref, wqk_ref, wv_ref, qv_ref):
    x = x_ref[...]
    m = jnp.mean(x, axis=-1, keepdims=True)
    var = jnp.mean((x - m) * (x - m), axis=-1, keepdims=True)
    h = (x - m) / jnp.sqrt(var + 1e-5) * s_ref[...][None, :] + b_ref[...][None, :]
    for hh in range(_H):
        sl = slice(hh * _DH, (hh + 1) * _DH)
        qv_ref[hh] = jnp.concatenate([h @ wqk_ref[:, sl], h @ wv_ref[:, sl]],
                                     axis=-1)


def _proj(x, s, b, wqk_l, wv_l):
    return pl.pallas_call(
        _proj_body,
        grid=(_S // _TS,),
        in_specs=[
            pl.BlockSpec((_TS, _D), lambda i: (i, 0)),
            pl.BlockSpec((_D,), lambda i: (0,)),
            pl.BlockSpec((_D,), lambda i: (0,)),
            pl.BlockSpec((_D, _D), lambda i: (0, 0)),
            pl.BlockSpec((_D, _D), lambda i: (0, 0)),
        ],
        out_specs=pl.BlockSpec((_H, _TS, 2 * _DH), lambda i: (0, i, 0)),
        out_shape=jax.ShapeDtypeStruct((_H, _S, 2 * _DH), _f32),
    )(x, s, b, wqk_l, wv_l)


def _bucket_body(qv_ref, rot_ref, out_ref):
    t = pl.program_id(0)
    qk = qv_ref[0][:, :_DH]
    rr = rot_ref[0]
    proj = lax.dot_general(qk, rr, (((1,), (0,)), ((), ())))
    x = jnp.concatenate([proj, -proj], axis=-1)

    r1 = lax.broadcasted_iota(jnp.int32, (_NB, _NB), 0)
    c1 = lax.broadcasted_iota(jnp.int32, (_NB, _NB), 1)
    lincl = (r1 >= c1).astype(_f32)
    ustrict = (r1 < c1).astype(_f32)
    mx = jnp.max(x, axis=-1, keepdims=True)
    eq = (x >= mx).astype(_f32)
    nleft = lax.dot_general(eq, ustrict, (((1,), (0,)), ((), ())))
    oh = eq * (nleft == 0.0).astype(_f32)

    nch = _S // _NB
    totals = jnp.sum(oh.reshape(nch, _NB, _NB), axis=1)
    r2 = lax.broadcasted_iota(jnp.int32, (nch, nch), 0)
    c2 = lax.broadcasted_iota(jnp.int32, (nch, nch), 1)
    lstrict = (r2 > c2).astype(_f32)
    chunk_excl = lax.dot_general(lstrict, totals, (((1,), (0,)), ((), ())))
    gcounts = chunk_excl[-1:] + totals[-1:]
    starts = lax.dot_general(gcounts, ustrict, (((1,), (0,)), ((), ())))
    base = starts + chunk_excl
    for c in range(nch):
        blk = oh[c * _NB:(c + 1) * _NB]
        incl = lax.dot_general(lincl, blk, (((1,), (0,)), ((), ())))
        dest_c = jnp.sum((incl + base[c:c + 1]) * blk, axis=-1) - 1.0
        out_ref[c] = dest_c.astype(jnp.int32) + t * _S


def _bucket(qv, rot_l):
    return pl.pallas_call(
        _bucket_body,
        grid=(_NT,),
        in_specs=[
            pl.BlockSpec((1, _S, 2 * _DH), lambda t: (t % _H, 0, 0)),
            pl.BlockSpec((1, _DH, _NB // 2), lambda t: (t // _H, 0, 0)),
        ],
        out_specs=pl.BlockSpec((_S // _NB, _NB), lambda t: (t, 0)),
        out_shape=jax.ShapeDtypeStruct((_NT * _S // _NB, _NB), jnp.int32),
    )(qv, rot_l)


def _sc_scatter_body(qv_hbm, gd_hbm, tab, postab,
                     idx0, idx1, idx2, idx3,
                     row0, row1, row2, row3,
                     pos0, pos1, pos2, pos3,
                     ss0, ss1, ss2, ss3, sc0, sc1, sc2, sc3):
    idxb = [idx0, idx1, idx2, idx3]
    rowb = [row0, row1, row2, row3]
    posb = [pos0, pos1, pos2, pos3]
    sstg = [ss0, ss1, ss2, ss3]
    ssct = [sc0, sc1, sc2, sc3]
    wid = lax.axis_index("s") * 2 + lax.axis_index("c")

    def chunk_addr(i):
        t = i // 2
        cc = wid * 2 + (i % 2)
        s0 = cc * 128
        return t * _S + s0, (t % _H) * _S + s0, s0

    stage_d = {}
    scat_d = {}

    def fire_stage(i):
        slot = i % 4
        g0, src0, s0 = chunk_addr(i)
        d1 = pltpu.async_copy(gd_hbm.at[pl.ds(g0, 128)], idxb[slot], sstg[slot])
        d2 = pltpu.async_copy(qv_hbm.at[pl.ds(src0, 128)], rowb[slot], sstg[slot])
        for k in range(8):
            sl = pl.ds(k * 16, 16)
            posb[slot][sl] = (lax.iota(jnp.int32, 16) + (s0 + k * 16)).astype(_f32)
        stage_d[i] = (d1, d2)

    fire_stage(0)
    fire_stage(1)
    for i in range(_CPW):
        slot = i % 4
        if i + 2 < _CPW:
            if i - 2 >= 0:
                for d in scat_d.pop(i - 2):
                    d.wait()
            fire_stage(i + 2)
        for d in stage_d.pop(i):
            d.wait()
        d3 = pltpu.async_copy(rowb[slot], tab.at[idxb[slot]], ssct[slot])
        d4 = pltpu.async_copy(posb[slot], postab.at[idxb[slot]], ssct[slot])
        scat_d[i] = (d3, d4)
    for i in sorted(scat_d):
        for d in scat_d[i]:
            d.wait()


def _sc_scatter(qvflat, gd):
    sem = pltpu.SemaphoreType.DMA
    return pl.kernel(
        _sc_scatter_body,
        out_type=(jax.ShapeDtypeStruct((_NTS, 2 * _DH), _f32),
                  jax.ShapeDtypeStruct((_NTS,), _f32)),
        mesh=plsc.VectorSubcoreMesh(core_axis_name="c", subcore_axis_name="s"),
        scratch_types=[pltpu.VMEM((128,), jnp.int32)] * 4
                      + [pltpu.VMEM((128, 2 * _DH), _f32)] * 4
                      + [pltpu.VMEM((128,), _f32)] * 4
                      + [sem] * 8,
        compiler_params=pltpu.CompilerParams(use_tc_tiling_on_sc=False),
    )(qvflat, gd)


def _attn_body(qc_ref, qp_ref, pc_ref, pp_ref, o_ref):
    cur = qc_ref[0]
    full = jnp.concatenate([qp_ref[0], cur], axis=0)
    kw = full[:, :_DH]
    vw = full[:, _DH:]
    kn = kw / (jnp.sqrt(jnp.sum(kw * kw, axis=-1, keepdims=True)) + 1e-6)
    q = cur[:, :_DH] * (1.0 / math.sqrt(_DH))
    pq = pc_ref[0]
    w = _TS + _C
    pfull = jnp.concatenate([pp_ref[0], pq], axis=0)
    r0 = lax.broadcasted_iota(jnp.int32, (w, w), 0)
    c0 = lax.broadcasted_iota(jnp.int32, (w, w), 1)
    eye = (r0 == c0).astype(_f32)
    pk = lax.dot_general(pfull, eye, (((0,), (0,)), ((), ())))
    sdiv = lax.broadcasted_iota(jnp.int32, (_TS, w), 0) // _C
    jdiv = lax.broadcasted_iota(jnp.int32, (_TS, w), 1) // _C
    win = (jdiv == sdiv) | (jdiv == sdiv + 1)
    dots = lax.dot_general(q, kn, (((1,), (1,)), ((), ())))
    dots = jnp.where(win & (pq >= pk), dots, -1e9)
    dots = jnp.where(win & (pq == pk), dots - 1e5, dots)
    m = jnp.max(dots, axis=-1, keepdims=True)
    e = jnp.exp(dots - m)
    ssum = jnp.sum(e, axis=-1, keepdims=True)
    lse = m + jnp.log(ssum)
    o = lax.dot_general(e / ssum, vw, (((1,), (0,)), ((), ())))
    pad = jnp.zeros((_TS, 2 * _DH - _DH - 1), _f32)
    o_ref[0] = jnp.concatenate([o, lse, pad], axis=-1)


def _attn(qvtab, postab):
    prev_map = lambda t, i: (t, (i * _NSUB + _NCH - 1) % _NCH, 0)
    return pl.pallas_call(
        _attn_body,
        grid=(_NT, _S // _TS),
        in_specs=[
            pl.BlockSpec((1, _TS, 2 * _DH), lambda t, i: (t, i, 0)),
            pl.BlockSpec((1, _C, 2 * _DH), prev_map),
            pl.BlockSpec((1, _TS, 1), lambda t, i: (t, i, 0)),
            pl.BlockSpec((1, _C, 1), prev_map),
        ],
        out_specs=pl.BlockSpec((1, _TS, 2 * _DH), lambda t, i: (t, i, 0)),
        out_shape=jax.ShapeDtypeStruct((_NT, _S, 2 * _DH), _f32),
    )(qvtab, qvtab, postab, postab)


def _sc_gather_body(otab, gd_hbm, ou,
                    idx0, idx1, idx2, idx3,
                    row0, row1, row2, row3,
                    si0, si1, si2, si3, sg0, sg1, sg2, sg3,
                    sw0, sw1, sw2, sw3):
    idxb = [idx0, idx1, idx2, idx3]
    rowb = [row0, row1, row2, row3]
    sidx = [si0, si1, si2, si3]
    sgat = [sg0, sg1, sg2, sg3]
    swrt = [sw0, sw1, sw2, sw3]
    wid = lax.axis_index("s") * 2 + lax.axis_index("c")

    def g_of(i):
        return (i // 2) * _S + (wid * 2 + (i % 2)) * 128

    idx_d = {}
    gat_d = {}
    wrt_d = {}

    def fire_idx(i):
        slot = i % 4
        idx_d[i] = pltpu.async_copy(gd_hbm.at[pl.ds(g_of(i), 128)],
                                    idxb[slot], sidx[slot])

    fire_idx(0)
    fire_idx(1)
    for i in range(_CPW):
        slot = i % 4
        if i + 2 < _CPW:
            if i - 2 >= 0:
                wrt_d.pop(i - 2).wait()
            fire_idx(i + 2)
        idx_d.pop(i).wait()
        gat_d[i] = pltpu.async_copy(otab.at[idxb[slot]], rowb[slot], sgat[slot])
        if i - 1 >= 0:
            gat_d.pop(i - 1).wait()
            ps = (i - 1) % 4
            wrt_d[i - 1] = pltpu.async_copy(rowb[ps], ou.at[pl.ds(g_of(i - 1), 128)],
                                            swrt[ps])
    gat_d.pop(_CPW - 1).wait()
    ls = (_CPW - 1) % 4
    wrt_d[_CPW - 1] = pltpu.async_copy(rowb[ls], ou.at[pl.ds(g_of(_CPW - 1), 128)],
                                       swrt[ls])
    for i in sorted(wrt_d):
        wrt_d[i].wait()


def _sc_gather(table, idx):
    sem = pltpu.SemaphoreType.DMA
    return pl.kernel(
        _sc_gather_body,
        out_type=jax.ShapeDtypeStruct((_NTS, 2 * _DH), _f32),
        mesh=plsc.VectorSubcoreMesh(core_axis_name="c", subcore_axis_name="s"),
        scratch_types=[pltpu.VMEM((128,), jnp.int32)] * 4
                      + [pltpu.VMEM((128, 2 * _DH), _f32)] * 4
                      + [sem] * 12,
        compiler_params=pltpu.CompilerParams(use_tc_tiling_on_sc=False),
    )(table, idx)


def _combine_body(o0_ref, o1_ref, x1_ref, wo_ref, y_ref):
    a0 = o0_ref[0]
    a1 = o1_ref[0]
    l0 = a0[..., _DH:_DH + 1]
    l1 = a1[..., _DH:_DH + 1]
    m = jnp.maximum(l0, l1)
    e0 = jnp.exp(l0 - m)
    e1 = jnp.exp(l1 - m)
    inv = 1.0 / (e0 + e1)
    acc = x1_ref[...]
    for hh in range(_H):
        ohh = ((e0[hh] * inv[hh]) * a0[hh, :, :_DH]
               + (e1[hh] * inv[hh]) * a1[hh, :, :_DH])
        acc = acc + lax.dot_general(
            ohh, wo_ref[hh * _DH:(hh + 1) * _DH, :], (((1,), (0,)), ((), ())))
    y_ref[...] = acc


def _combine(ou4, x1c, wo_l):
    return pl.pallas_call(
        _combine_body,
        grid=(_S // _TS,),
        in_specs=[
            pl.BlockSpec((1, _H, _TS, 2 * _DH), lambda i: (0, 0, i, 0)),
            pl.BlockSpec((1, _H, _TS, 2 * _DH), lambda i: (1, 0, i, 0)),
            pl.BlockSpec((_TS, _D), lambda i: (i, 0)),
            pl.BlockSpec((_D, _D), lambda i: (0, 0)),
        ],
        out_specs=pl.BlockSpec((_TS, _D), lambda i: (i, 0)),
        out_shape=jax.ShapeDtypeStruct((_S, _D), _f32),
    )(ou4, ou4, x1c, wo_l)


def _ff_body(y1_ref, x2_ref, s_ref, b_ref, w1_ref, b1_ref, w2_ref, b2_ref,
             out_ref):
    x = y1_ref[...]
    m = jnp.mean(x, axis=-1, keepdims=True)
    var = jnp.mean((x - m) * (x - m), axis=-1, keepdims=True)
    h = (x - m) / jnp.sqrt(var + 1e-5) * s_ref[...][None, :] + b_ref[...][None, :]
    a = h @ w1_ref[...] + b1_ref[...][None, :]
    g = jax.nn.gelu(a)
    out_ref[...] = x2_ref[...] + g @ w2_ref[...] + b2_ref[...][None, :]


def _ff(y1, x2c, s, b, w1_l, b1_l, w2_l, b2_l):
    return pl.pallas_call(
        _ff_body,
        grid=(_S // _TS,),
        in_specs=[
            pl.BlockSpec((_TS, _D), lambda i: (i, 0)),
            pl.BlockSpec((_TS, _D), lambda i: (i, 0)),
            pl.BlockSpec((_D,), lambda i: (0,)),
            pl.BlockSpec((_D,), lambda i: (0,)),
            pl.BlockSpec((_D, _F), lambda i: (0, 0)),
            pl.BlockSpec((_F,), lambda i: (0,)),
            pl.BlockSpec((_F, _D), lambda i: (0, 0)),
            pl.BlockSpec((_D,), lambda i: (0,)),
        ],
        out_specs=pl.BlockSpec((_TS, _D), lambda i: (i, 0)),
        out_shape=jax.ShapeDtypeStruct((_S, _D), _f32),
    )(y1, x2c, s, b, w1_l, b1_l, w2_l, b2_l)


def kernel(x1, x2, ln1_s, ln1_b, wqk, wv, wo, rot, ln2_s, ln2_b, w1, b1, w2, b2):
    xa = x1[0]
    xb = x2[0]
    for l in range(2):
        qv = _proj(xb, ln1_s[l], ln1_b[l], wqk[l], wv[l])
        gd = _bucket(qv, rot[l]).reshape(_NTS)
        qvtab, postab = _sc_scatter(qv.reshape(_H * _S, 2 * _DH), gd)
        olu = _attn(qvtab.reshape(_NT, _S, 2 * _DH), postab.reshape(_NT, _S, 1))
        ou = _sc_gather(olu.reshape(_NTS, 2 * _DH), gd)
        ya = _combine(ou.reshape(_R, _H, _S, 2 * _DH), xa, wo[l])
        yb = _ff(ya, xb, ln2_s[l], ln2_b[l], w1[l], b1[l], w2[l], b2[l])
        xa, xb = ya, yb
    return xb[None]
